# baseline (speedup 1.0000x reference)
import functools

import jax
import jax.numpy as jnp
from jax import lax
from jax.experimental import pallas as pl
from jax.experimental.pallas import tpu as pltpu
from jax.experimental.pallas import tpu_sc as plsc

_B, _T, _P = 32, 4096, 128
_ROWS = _B * _T
_NC, _NS = 2, 16
_NW = _NC * _NS
_RPW = _ROWS // _NW
_C = 128
_CHUNK = _C * _P
_NCHUNK = _RPW // _C
_GROUPS = _C // 16
_NCHAIN = 8
_CLEN = _P // _NCHAIN


def _argmax_group(in_b, rowoff):
    iota = lax.iota(jnp.int32, 16)
    bases = [rowoff + _CLEN * t for t in range(_NCHAIN)]
    ms = []
    bis = []
    for t in range(_NCHAIN):
        idx = bases[t] + iota
        ms.append(plsc.load_gather(in_b, [idx]))
        bis.append(idx)

    def col_step(j, carry):
        cv = (carry[0] + 1) & (_CLEN - 1)
        out = [cv]
        for t in range(_NCHAIN):
            m, bi = carry[1 + 2 * t], carry[2 + 2 * t]
            idx = bases[t] + cv
            v = plsc.load_gather(in_b, [idx])
            gt = v > m
            out.append(jnp.where(gt, v, m))
            out.append(jnp.where(gt, idx, bi))
        return tuple(out)

    init = [iota]
    for t in range(_NCHAIN):
        init.append(ms[t])
        init.append(bis[t])
    res = plsc.parallel_loop(1, _CLEN, unroll=15, carry=tuple(init))(col_step)
    ms = [res[1 + 2 * t] for t in range(_NCHAIN)]
    bis = [res[2 + 2 * t] for t in range(_NCHAIN)]
    while len(ms) > 1:
        nm, nb = [], []
        for i in range(0, len(ms), 2):
            ge = ms[i] >= ms[i + 1]
            nm.append(jnp.where(ge, ms[i], ms[i + 1]))
            nb.append(jnp.where(ge, bis[i], bis[i + 1]))
        ms, bis = nm, nb
    return bis[0]


def _sc_body(x_hbm, o_hbm, in0, in1, out0, out1, pos0, pos1,
             isem0, isem1, osem0, osem1):
    wid = lax.axis_index("s") * _NC + lax.axis_index("c")
    base = wid * (_RPW * _P)

    iota = lax.iota(jnp.int32, 16)
    rowoff0 = iota * _P
    zero16 = jnp.zeros((16,), jnp.float32)
    one16 = jnp.ones((16,), jnp.float32)

    ins = (in0, in1)
    outs = (out0, out1)
    poss = (pos0, pos1)
    isems = (isem0, isem1)
    osems = (osem0, osem1)

    def zinit(i, carry):
        out0[pl.ds(i * 16, 16)] = zero16
        out1[pl.ds(i * 16, 16)] = zero16
        return carry
    lax.fori_loop(0, _CHUNK // 16, zinit, 0)
    for g in range(_GROUPS):
        pos0[pl.ds(g * 16, 16)] = rowoff0 + g * (16 * _P)
        pos1[pl.ds(g * 16, 16)] = rowoff0 + g * (16 * _P)

    pltpu.async_copy(x_hbm.at[pl.ds(base, _CHUNK)], in0, isem0)
    pltpu.async_copy(x_hbm.at[pl.ds(base + _CHUNK, _CHUNK)], in1, isem1)

    def step_fn(step, carry):
        for b in range(2):
            chunk = step * 2 + b
            off = base + chunk * _CHUNK
            in_b, out_b, pos_b = ins[b], outs[b], poss[b]
            isem, osem = isems[b], osems[b]

            pltpu.make_async_copy(
                x_hbm.at[pl.ds(off, _CHUNK)], in_b, isem).wait()

            @pl.when(step > 0)
            def _wait_prev_out():
                poff = base + (chunk - 2) * _CHUNK
                pltpu.make_async_copy(
                    out_b, o_hbm.at[pl.ds(poff, _CHUNK)], osem).wait()

            def group_fn(g, gcarry):
                rowoff = rowoff0 + g * (16 * _P)
                bi = _argmax_group(in_b, rowoff)
                prev = pos_b[pl.ds(g * 16, 16)]
                plsc.store_scatter(out_b, [prev], zero16, mask=prev != bi)
                plsc.store_scatter(out_b, [bi], one16)
                pos_b[pl.ds(g * 16, 16)] = bi
                return gcarry
            lax.fori_loop(0, _GROUPS, group_fn, 0)

            pltpu.async_copy(out_b, o_hbm.at[pl.ds(off, _CHUNK)], osem)

            @pl.when(step < _NCHUNK // 2 - 1)
            def _fetch_next():
                noff = base + (chunk + 2) * _CHUNK
                pltpu.async_copy(x_hbm.at[pl.ds(noff, _CHUNK)], in_b, isem)
        return carry

    lax.fori_loop(0, _NCHUNK // 2, step_fn, 0)

    for b in range(2):
        off = base + (_NCHUNK - 2 + b) * _CHUNK
        pltpu.make_async_copy(
            outs[b], o_hbm.at[pl.ds(off, _CHUNK)], osems[b]).wait()


@functools.partial(jax.jit, static_argnums=())
def _sc_onehot_argmax(xf):
    mesh = plsc.VectorSubcoreMesh(
        core_axis_name="c", subcore_axis_name="s",
        num_cores=_NC, num_subcores=_NS)
    f = pl.kernel(
        _sc_body,
        out_type=jax.ShapeDtypeStruct((_ROWS * _P,), jnp.float32),
        mesh=mesh,
        scratch_types=[
            pltpu.VMEM((_CHUNK,), jnp.float32),
            pltpu.VMEM((_CHUNK,), jnp.float32),
            pltpu.VMEM((_CHUNK,), jnp.float32),
            pltpu.VMEM((_CHUNK,), jnp.float32),
            pltpu.VMEM((_C,), jnp.int32),
            pltpu.VMEM((_C,), jnp.int32),
            pltpu.SemaphoreType.DMA,
            pltpu.SemaphoreType.DMA,
            pltpu.SemaphoreType.DMA,
            pltpu.SemaphoreType.DMA,
        ],
        compiler_params=pltpu.CompilerParams(needs_layout_passes=False),
    )
    return f(xf)


def kernel(x):
    b, t, p = x.shape
    y = _sc_onehot_argmax(x.reshape(-1))
    return y.reshape(b, t, p)


# speedup vs baseline: 2.0851x; 2.0851x over previous
import functools

import jax
import jax.numpy as jnp
from jax import lax
from jax.experimental import pallas as pl
from jax.experimental.pallas import tpu as pltpu
from jax.experimental.pallas import tpu_sc as plsc

_B, _T, _P = 32, 4096, 128
_ROWS = _B * _T
_NC, _NS = 2, 16
_NW = _NC * _NS
_RPW = _ROWS // _NW
_C = 128
_CHUNK = _C * _P
_NCHUNK = _RPW // _C
_GROUPS = _C // 16
_NCHAIN = 8
_CLEN = _P // _NCHAIN


def _argmax_group(in_b, rowoff):
    iota = lax.iota(jnp.int32, 16)
    bases = [rowoff + _CLEN * t for t in range(_NCHAIN)]
    ms = []
    bis = []
    for t in range(_NCHAIN):
        idx = bases[t] + iota
        ms.append(plsc.load_gather(in_b, [idx]))
        bis.append(idx)

    def col_step(j, carry):
        cv = (carry[0] + 1) & (_CLEN - 1)
        out = [cv]
        for t in range(_NCHAIN):
            m, bi = carry[1 + 2 * t], carry[2 + 2 * t]
            idx = bases[t] + cv
            v = plsc.load_gather(in_b, [idx])
            gt = v > m
            out.append(jnp.where(gt, v, m))
            out.append(jnp.where(gt, idx, bi))
        return tuple(out)

    init = [iota]
    for t in range(_NCHAIN):
        init.append(ms[t])
        init.append(bis[t])
    res = plsc.parallel_loop(1, _CLEN, unroll=5, carry=tuple(init))(col_step)
    ms = [res[1 + 2 * t] for t in range(_NCHAIN)]
    bis = [res[2 + 2 * t] for t in range(_NCHAIN)]
    while len(ms) > 1:
        nm, nb = [], []
        for i in range(0, len(ms), 2):
            ge = ms[i] >= ms[i + 1]
            nm.append(jnp.where(ge, ms[i], ms[i + 1]))
            nb.append(jnp.where(ge, bis[i], bis[i + 1]))
        ms, bis = nm, nb
    return bis[0]


def _sc_body(x_hbm, o_hbm, in0, in1, in2, out0, out1, pos0, pos1,
             isem0, isem1, isem2, osem0, osem1):
    wid = lax.axis_index("s") * _NC + lax.axis_index("c")
    base = wid * (_RPW * _P)

    iota = lax.iota(jnp.int32, 16)
    rowoff0 = iota * _P
    zero16 = jnp.zeros((16,), jnp.float32)
    one16 = jnp.ones((16,), jnp.float32)

    ins = (in0, in1, in2)
    outs = (out0, out1)
    poss = (pos0, pos1)
    isems = (isem0, isem1, isem2)
    osems = (osem0, osem1)

    def zinit(i, carry):
        out0[pl.ds(i * 16, 16)] = zero16
        out1[pl.ds(i * 16, 16)] = zero16
        return carry
    lax.fori_loop(0, _CHUNK // 16, zinit, 0)
    for g in range(_GROUPS):
        pos0[pl.ds(g * 16, 16)] = rowoff0 + g * (16 * _P)
        pos1[pl.ds(g * 16, 16)] = rowoff0 + g * (16 * _P)

    pltpu.async_copy(x_hbm.at[pl.ds(base, _CHUNK)], in0, isem0)
    pltpu.async_copy(x_hbm.at[pl.ds(base + _CHUNK, _CHUNK)], in1, isem1)
    pltpu.async_copy(x_hbm.at[pl.ds(base + 2 * _CHUNK, _CHUNK)], in2, isem2)

    def do_chunk(chunk, bi3, bo2, last):
        off = base + chunk * _CHUNK
        in_b, out_b, pos_b = ins[bi3], outs[bo2], poss[bo2]
        isem, osem = isems[bi3], osems[bo2]

        pltpu.make_async_copy(
            x_hbm.at[pl.ds(off, _CHUNK)], in_b, isem).wait()

        @pl.when(chunk >= 2)
        def _wait_prev_out():
            poff = base + (chunk - 2) * _CHUNK
            pltpu.make_async_copy(
                out_b, o_hbm.at[pl.ds(poff, _CHUNK)], osem).wait()

        def group_fn(g, gcarry):
            rowoff = rowoff0 + g * (16 * _P)
            bi = _argmax_group(in_b, rowoff)
            prev = pos_b[pl.ds(g * 16, 16)]
            plsc.store_scatter(out_b, [prev], zero16, mask=prev != bi)
            plsc.store_scatter(out_b, [bi], one16)
            pos_b[pl.ds(g * 16, 16)] = bi
            return gcarry
        lax.fori_loop(0, _GROUPS, group_fn, 0)

        pltpu.async_copy(out_b, o_hbm.at[pl.ds(off, _CHUNK)], osem)

        if not last:
            @pl.when(chunk + 3 < _NCHUNK)
            def _fetch_next():
                noff = base + (chunk + 3) * _CHUNK
                pltpu.async_copy(x_hbm.at[pl.ds(noff, _CHUNK)], in_b, isem)

    def super_step(s, carry):
        c0 = s * 6
        for k in range(6):
            do_chunk(c0 + k, k % 3, k % 2, False)
        return carry

    lax.fori_loop(0, (_NCHUNK - 2) // 6, super_step, 0)
    do_chunk(_NCHUNK - 2, (_NCHUNK - 2) % 3, 0, True)
    do_chunk(_NCHUNK - 1, (_NCHUNK - 1) % 3, 1, True)

    for b in range(2):
        off = base + (_NCHUNK - 2 + b) * _CHUNK
        pltpu.make_async_copy(
            outs[b], o_hbm.at[pl.ds(off, _CHUNK)], osems[b]).wait()


@functools.partial(jax.jit, static_argnums=())
def _sc_onehot_argmax(xf):
    mesh = plsc.VectorSubcoreMesh(
        core_axis_name="c", subcore_axis_name="s",
        num_cores=_NC, num_subcores=_NS)
    f = pl.kernel(
        _sc_body,
        out_type=jax.ShapeDtypeStruct((_ROWS * _P,), jnp.float32),
        mesh=mesh,
        scratch_types=[
            pltpu.VMEM((_CHUNK,), jnp.float32),
            pltpu.VMEM((_CHUNK,), jnp.float32),
            pltpu.VMEM((_CHUNK,), jnp.float32),
            pltpu.VMEM((_CHUNK,), jnp.float32),
            pltpu.VMEM((_CHUNK,), jnp.float32),
            pltpu.VMEM((_C,), jnp.int32),
            pltpu.VMEM((_C,), jnp.int32),
            pltpu.SemaphoreType.DMA,
            pltpu.SemaphoreType.DMA,
            pltpu.SemaphoreType.DMA,
            pltpu.SemaphoreType.DMA,
            pltpu.SemaphoreType.DMA,
        ],
        compiler_params=pltpu.CompilerParams(needs_layout_passes=False),
    )
    return f(xf)


def kernel(x):
    b, t, p = x.shape
    y = _sc_onehot_argmax(x.reshape(-1))
    return y.reshape(b, t, p)


# speedup vs baseline: 2.1209x; 1.0172x over previous
import functools

import jax
import jax.numpy as jnp
from jax import lax
from jax.experimental import pallas as pl
from jax.experimental.pallas import tpu as pltpu
from jax.experimental.pallas import tpu_sc as plsc

_B, _T, _P = 32, 4096, 128
_ROWS = _B * _T
_NC, _NS = 2, 16
_NW = _NC * _NS
_RPW = _ROWS // _NW
_C = 128
_CHUNK = _C * _P
_NCHUNK = _RPW // _C
_GROUPS = _C // 16
_NCHAIN = 8
_CLEN = _P // _NCHAIN


def _argmax_group(in_b, rowoff):
    iota = lax.iota(jnp.int32, 16)
    bases = [rowoff + _CLEN * t for t in range(_NCHAIN)]
    ms = []
    bis = []
    for t in range(_NCHAIN):
        idx = bases[t] + iota
        ms.append(plsc.load_gather(in_b, [idx]))
        bis.append(idx)

    def col_step(j, carry):
        cv = (carry[0] + 1) & (_CLEN - 1)
        out = [cv]
        for t in range(_NCHAIN):
            m, bi = carry[1 + 2 * t], carry[2 + 2 * t]
            idx = bases[t] + cv
            v = plsc.load_gather(in_b, [idx])
            gt = v > m
            out.append(jnp.where(gt, v, m))
            out.append(jnp.where(gt, idx, bi))
        return tuple(out)

    init = [iota]
    for t in range(_NCHAIN):
        init.append(ms[t])
        init.append(bis[t])
    res = plsc.parallel_loop(1, _CLEN, unroll=5, carry=tuple(init))(col_step)
    ms = [res[1 + 2 * t] for t in range(_NCHAIN)]
    bis = [res[2 + 2 * t] for t in range(_NCHAIN)]
    while len(ms) > 1:
        nm, nb = [], []
        for i in range(0, len(ms), 2):
            ge = ms[i] >= ms[i + 1]
            nm.append(jnp.where(ge, ms[i], ms[i + 1]))
            nb.append(jnp.where(ge, bis[i], bis[i + 1]))
        ms, bis = nm, nb
    return bis[0]


def _sc_body(x_hbm, o_hbm, in0, in1, in2, out0, out1, out2, pos0, pos1,
             pos2, isem0, isem1, isem2, osem0, osem1, osem2):
    wid = lax.axis_index("s") * _NC + lax.axis_index("c")
    base = wid * (_RPW * _P)

    iota = lax.iota(jnp.int32, 16)
    rowoff0 = iota * _P
    zero16 = jnp.zeros((16,), jnp.float32)
    one16 = jnp.ones((16,), jnp.float32)

    ins = (in0, in1, in2)
    outs = (out0, out1, out2)
    poss = (pos0, pos1, pos2)
    isems = (isem0, isem1, isem2)
    osems = (osem0, osem1, osem2)

    def zinit(i, carry):
        out0[pl.ds(i * 16, 16)] = zero16
        out1[pl.ds(i * 16, 16)] = zero16
        out2[pl.ds(i * 16, 16)] = zero16
        return carry
    lax.fori_loop(0, _CHUNK // 16, zinit, 0)
    for g in range(_GROUPS):
        pos0[pl.ds(g * 16, 16)] = rowoff0 + g * (16 * _P)
        pos1[pl.ds(g * 16, 16)] = rowoff0 + g * (16 * _P)
        pos2[pl.ds(g * 16, 16)] = rowoff0 + g * (16 * _P)

    pltpu.async_copy(x_hbm.at[pl.ds(base, _CHUNK)], in0, isem0)
    pltpu.async_copy(x_hbm.at[pl.ds(base + _CHUNK, _CHUNK)], in1, isem1)
    pltpu.async_copy(x_hbm.at[pl.ds(base + 2 * _CHUNK, _CHUNK)], in2, isem2)

    def do_chunk(chunk, b3, last):
        off = base + chunk * _CHUNK
        in_b, out_b, pos_b = ins[b3], outs[b3], poss[b3]
        isem, osem = isems[b3], osems[b3]

        pltpu.make_async_copy(
            x_hbm.at[pl.ds(off, _CHUNK)], in_b, isem).wait()

        @pl.when(chunk >= 3)
        def _wait_prev_out():
            poff = base + (chunk - 3) * _CHUNK
            pltpu.make_async_copy(
                out_b, o_hbm.at[pl.ds(poff, _CHUNK)], osem).wait()

        def group_fn(g, gcarry):
            rowoff = rowoff0 + g * (16 * _P)
            bi = _argmax_group(in_b, rowoff)
            prev = pos_b[pl.ds(g * 16, 16)]
            plsc.store_scatter(out_b, [prev], zero16, mask=prev != bi)
            plsc.store_scatter(out_b, [bi], one16)
            pos_b[pl.ds(g * 16, 16)] = bi
            return gcarry
        lax.fori_loop(0, _GROUPS, group_fn, 0)

        pltpu.async_copy(out_b, o_hbm.at[pl.ds(off, _CHUNK)], osem)

        if not last:
            @pl.when(chunk + 3 < _NCHUNK)
            def _fetch_next():
                noff = base + (chunk + 3) * _CHUNK
                pltpu.async_copy(x_hbm.at[pl.ds(noff, _CHUNK)], in_b, isem)


    def super_step(s, carry):
        c0 = s * 3
        for k in range(3):
            do_chunk(c0 + k, k, False)
        return carry

    lax.fori_loop(0, (_NCHUNK - 2) // 3, super_step, 0)
    do_chunk(_NCHUNK - 2, (_NCHUNK - 2) % 3, True)
    do_chunk(_NCHUNK - 1, (_NCHUNK - 1) % 3, True)

    for chunk in range(_NCHUNK - 3, _NCHUNK):
        off = base + chunk * _CHUNK
        pltpu.make_async_copy(
            outs[chunk % 3], o_hbm.at[pl.ds(off, _CHUNK)],
            osems[chunk % 3]).wait()


@functools.partial(jax.jit, static_argnums=())
def _sc_onehot_argmax(xf):
    mesh = plsc.VectorSubcoreMesh(
        core_axis_name="c", subcore_axis_name="s",
        num_cores=_NC, num_subcores=_NS)
    f = pl.kernel(
        _sc_body,
        out_type=jax.ShapeDtypeStruct((_ROWS * _P,), jnp.float32),
        mesh=mesh,
        scratch_types=[
            pltpu.VMEM((_CHUNK,), jnp.float32),
            pltpu.VMEM((_CHUNK,), jnp.float32),
            pltpu.VMEM((_CHUNK,), jnp.float32),
            pltpu.VMEM((_CHUNK,), jnp.float32),
            pltpu.VMEM((_CHUNK,), jnp.float32),
            pltpu.VMEM((_CHUNK,), jnp.float32),
            pltpu.VMEM((_C,), jnp.int32),
            pltpu.VMEM((_C,), jnp.int32),
            pltpu.VMEM((_C,), jnp.int32),
            pltpu.SemaphoreType.DMA,
            pltpu.SemaphoreType.DMA,
            pltpu.SemaphoreType.DMA,
            pltpu.SemaphoreType.DMA,
            pltpu.SemaphoreType.DMA,
            pltpu.SemaphoreType.DMA,
        ],
        compiler_params=pltpu.CompilerParams(needs_layout_passes=False),
    )
    return f(xf)


def kernel(x):
    b, t, p = x.shape
    y = _sc_onehot_argmax(x.reshape(-1))
    return y.reshape(b, t, p)
